# Initial kernel scaffold; baseline (speedup 1.0000x reference)
import jax
import jax.numpy as jnp
from jax.experimental import pallas as pl


def kernel(h, edge_index, edge_weight, W_emb, b_emb, Wp0, bp0, Wg0, mu0, sigma0, root0, bias0, Wp1, bp1, Wg1, mu1, sigma1, root1, bias1, Wm, bm):
    raise NotImplementedError("write your pallas kernel here")


# speedup vs baseline: 9.1400x; 9.1400x over previous
import functools

import jax
import jax.numpy as jnp
from jax import lax
from jax.experimental import pallas as pl
from jax.experimental.pallas import tpu as pltpu
from jax.experimental.pallas import tpu_sc as plsc

N = 10000
E = 320000
F = 128
EPS = 1e-15

NC = 2
NS = 16
NW = NC * NS

N_PAD = 10240
RPT = N_PAD // NS
EPT = E // NW
CH = 80
NCHUNK = EPT // CH
DEG_LANES = 16

_MESH = dict(core_axis_name="c", subcore_axis_name="s", num_cores=NC,
             num_subcores=NS)

BLK = 2000
GRID = N // BLK


def _m0_body(h_ref, wemb_ref, bemb_ref, wg_ref, root_ref, bias_ref,
             h1_ref, xg_ref, hr_ref):
  h1 = jnp.dot(h_ref[...], wemb_ref[...],
               preferred_element_type=jnp.float32) + bemb_ref[...]
  h1_ref[...] = h1
  xg_ref[...] = jnp.dot(h1, wg_ref[...], preferred_element_type=jnp.float32)
  hr_ref[...] = jnp.dot(h1, root_ref[...],
                        preferred_element_type=jnp.float32) + bias_ref[...]


def _m1_body(h1_ref, ap0_ref, ap1_ref, hr_ref, wg_ref, root_ref, bias_ref,
             h2_ref, xg_ref, hr1_ref):
  aggr = ap0_ref[...] + ap1_ref[...]
  h2 = h1_ref[...] + jax.nn.relu(aggr + hr_ref[...])
  h2_ref[...] = h2
  xg_ref[...] = jnp.dot(h2, wg_ref[...], preferred_element_type=jnp.float32)
  hr1_ref[...] = jnp.dot(h2, root_ref[...],
                         preferred_element_type=jnp.float32) + bias_ref[...]


def _m2_body(h2_ref, ap0_ref, ap1_ref, hr_ref, wm_ref, bm_ref, o_ref):
  aggr = ap0_ref[...] + ap1_ref[...]
  h3 = h2_ref[...] + jax.nn.relu(aggr + hr_ref[...])
  o_ref[...] = jnp.dot(h3, wm_ref[...],
                       preferred_element_type=jnp.float32) + bm_ref[...]


def _row_spec():
  return pl.BlockSpec((BLK, F), lambda i: (i, 0))


def _full_spec(shape):
  nd = len(shape)
  return pl.BlockSpec(shape, lambda i: (0,) * nd)


def _part_spec():
  return pl.BlockSpec((NC, BLK, F), lambda i: (0, i, 0))


def _deg_body(row_hbm, ew_hbm, out0_hbm, out1_hbm, zbuf, row_v, ew_v, accum):
  cid = lax.axis_index("c")
  sid = lax.axis_index("s")
  wid = cid * NS + sid
  zero = jnp.zeros((16,), jnp.float32)

  def zrow(i, _):
    zbuf[pl.ds(i * 16, 16)] = zero
    return 0

  lax.fori_loop(0, RPT // 16, zrow, 0)
  pltpu.sync_copy(zbuf, accum.at[pl.ds(sid * RPT, RPT)])
  plsc.subcore_barrier()

  ebase = wid * EPT

  def chunk(c, _):
    base = ebase + c * CH
    pltpu.sync_copy(row_hbm.at[pl.ds(base, CH)], row_v)
    pltpu.sync_copy(ew_hbm.at[pl.ds(base, CH)], ew_v)
    pltpu.sync_copy(ew_v, accum.at[row_v], add=True)
    return 0

  lax.fori_loop(0, NCHUNK, chunk, 0)
  plsc.subcore_barrier()

  @pl.when(cid == 0)
  def _():
    pltpu.sync_copy(accum.at[pl.ds(sid * RPT, RPT)],
                    out0_hbm.at[pl.ds(sid * RPT, RPT)])

  @pl.when(cid == 1)
  def _():
    pltpu.sync_copy(accum.at[pl.ds(sid * RPT, RPT)],
                    out1_hbm.at[pl.ds(sid * RPT, RPT)])


_NPW = N_PAD // NW


def _dis_body(deg0_hbm, deg1_hbm, dis_hbm, s0, s1, dv):
  cid = lax.axis_index("c")
  sid = lax.axis_index("s")
  wid = cid * NS + sid
  base = wid * _NPW
  pltpu.sync_copy(deg0_hbm.at[pl.ds(base, _NPW)], s0)
  pltpu.sync_copy(deg1_hbm.at[pl.ds(base, _NPW)], s1)
  half = jnp.full((16,), 0.5, jnp.float32)
  three_half = jnp.full((16,), 1.5, jnp.float32)
  magic = jnp.full((16,), 0x5F3759DF, jnp.int32)
  fzero = jnp.zeros((16,), jnp.float32)

  def grp(g, _):
    sl = pl.ds(g * 16, 16)
    d = s0[sl] + s1[sl]
    y = lax.bitcast_convert_type(
        magic - (lax.bitcast_convert_type(d, jnp.int32) >> 1), jnp.float32)
    hx = d * half
    y = y * (three_half - hx * y * y)
    y = y * (three_half - hx * y * y)
    y = y * (three_half - hx * y * y)
    dv[sl] = jnp.where(d > fzero, y, fzero)
    return 0

  lax.fori_loop(0, _NPW // 16, grp, 0)
  pltpu.sync_copy(dv, dis_hbm.at[pl.ds(base, _NPW)])


def _spmm_body(row_hbm, col_hbm, dis_hbm, p_hbm, xg_hbm,
               out0_hbm, out1_hbm,
               dis_s, p_v, row_v, col_v, dr_v, dc_v, gbuf, zbuf, accum):
  cid = lax.axis_index("c")
  sid = lax.axis_index("s")
  wid = cid * NS + sid
  zero = jnp.zeros((16,), jnp.float32)

  def zrow(i, _):
    for j in range(F // 16):
      zbuf[i, pl.ds(j * 16, 16)] = zero
    return 0

  lax.fori_loop(0, 128, zrow, 0)
  for t in range(RPT // 128):
    pltpu.sync_copy(zbuf, accum.at[pl.ds(sid * RPT + t * 128, 128)])

  @pl.when(sid == 0)
  def _():
    pltpu.sync_copy(dis_hbm, dis_s)

  pltpu.sync_copy(p_hbm, p_v)
  plsc.subcore_barrier()

  pvec = p_v[...]
  a0 = jnp.broadcast_to(lax.slice(pvec, (0,), (1,)), (16,))
  a1 = jnp.broadcast_to(lax.slice(pvec, (1,), (2,)), (16,))
  bb = jnp.broadcast_to(lax.slice(pvec, (2,), (3,)), (16,))
  mu = jnp.broadcast_to(lax.slice(pvec, (3,), (4,)), (16,))
  c2 = jnp.broadcast_to(lax.slice(pvec, (4,), (5,)), (16,))
  one = jnp.ones((16,), jnp.float32)
  two = jnp.full((16,), 2.0, jnp.float32)

  ebase = wid * EPT

  def chunk(c, _):
    base = ebase + c * CH
    pltpu.sync_copy(row_hbm.at[pl.ds(base, CH)], row_v)
    pltpu.sync_copy(col_hbm.at[pl.ds(base, CH)], col_v)
    pltpu.sync_copy(xg_hbm.at[row_v], gbuf)
    pltpu.sync_copy(dis_s.at[row_v], dr_v)
    pltpu.sync_copy(dis_s.at[col_v], dc_v)
    for g in range(CH // 16):
      x = a0 * dr_v[pl.ds(g * 16, 16)] + \
          a1 * dc_v[pl.ds(g * 16, 16)] + bb
      t = one - two / (jnp.exp(x + x) + one)
      d = t - mu
      w = jnp.exp(c2 * d * d)
      for i in range(16):
        wb = jnp.broadcast_to(lax.slice(w, (i,), (i + 1,)), (16,))
        r = g * 16 + i
        for j in range(F // 16):
          sl = pl.ds(j * 16, 16)
          gbuf[r, sl] = gbuf[r, sl] * wb
    pltpu.sync_copy(gbuf, accum.at[col_v], add=True)
    return 0

  lax.fori_loop(0, NCHUNK, chunk, 0)
  plsc.subcore_barrier()

  @pl.when(cid == 0)
  def _():
    pltpu.sync_copy(accum.at[pl.ds(sid * RPT, RPT)],
                    out0_hbm.at[pl.ds(sid * RPT, RPT)])

  @pl.when(cid == 1)
  def _():
    pltpu.sync_copy(accum.at[pl.ds(sid * RPT, RPT)],
                    out1_hbm.at[pl.ds(sid * RPT, RPT)])


@functools.lru_cache(maxsize=1)
def _sc_kernels():
  mesh = plsc.VectorSubcoreMesh(**_MESH)
  deg_k = pl.kernel(
      _deg_body,
      out_type=[jax.ShapeDtypeStruct((N_PAD,), jnp.float32)] * 2,
      mesh=mesh,
      scratch_types=[
          pltpu.VMEM((RPT,), jnp.float32),
          pltpu.VMEM((CH,), jnp.int32),
          pltpu.VMEM((CH,), jnp.float32),
          pltpu.VMEM_SHARED((N_PAD,), jnp.float32),
      ],
  )
  dis_k = pl.kernel(
      _dis_body,
      out_type=jax.ShapeDtypeStruct((N_PAD,), jnp.float32),
      mesh=mesh,
      scratch_types=[
          pltpu.VMEM((_NPW,), jnp.float32),
          pltpu.VMEM((_NPW,), jnp.float32),
          pltpu.VMEM((_NPW,), jnp.float32),
      ],
  )
  spmm_k = pl.kernel(
      _spmm_body,
      out_type=[jax.ShapeDtypeStruct((N_PAD, F), jnp.float32)] * 2,
      mesh=mesh,
      scratch_types=[
          pltpu.VMEM_SHARED((N_PAD,), jnp.float32),
          pltpu.VMEM((16,), jnp.float32),
          pltpu.VMEM((CH,), jnp.int32),
          pltpu.VMEM((CH,), jnp.int32),
          pltpu.VMEM((CH,), jnp.float32),
          pltpu.VMEM((CH,), jnp.float32),
          pltpu.VMEM((CH, F), jnp.float32),
          pltpu.VMEM((128, F), jnp.float32),
          pltpu.VMEM_SHARED((N_PAD, F), jnp.float32),
      ],
  )
  return deg_k, dis_k, spmm_k


def _params_vec(Wp, bp, mu, sigma):
  a0 = Wp[0, 0]
  a1 = Wp[1, 0]
  b = bp[0]
  m = mu[0, 0]
  c2 = -0.5 / (EPS + sigma[0, 0] ** 2)
  return jnp.stack([a0, a1, b, m, c2,
                    0., 0., 0., 0., 0., 0., 0., 0., 0., 0., 0.]
                   ).astype(jnp.float32)


def kernel(h, edge_index, edge_weight, W_emb, b_emb, Wp0, bp0, Wg0, mu0,
           sigma0, root0, bias0, Wp1, bp1, Wg1, mu1, sigma1, root1, bias1,
           Wm, bm):
  row = edge_index[0]
  col = edge_index[1]
  _deg_kernel, _dis_kernel, _spmm_kernel = _sc_kernels()

  deg0, deg1 = _deg_kernel(row, edge_weight)
  dis = _dis_kernel(deg0, deg1)

  p0 = _params_vec(Wp0, bp0, mu0, sigma0)
  p1 = _params_vec(Wp1, bp1, mu1, sigma1)

  m0 = pl.pallas_call(
      _m0_body,
      grid=(GRID,),
      in_specs=[_row_spec(), _full_spec((F, F)), _full_spec((1, F)),
                _full_spec((F, F)), _full_spec((F, F)), _full_spec((1, F))],
      out_specs=[_row_spec(), _row_spec(), _row_spec()],
      out_shape=[jax.ShapeDtypeStruct((N, F), jnp.float32)] * 3,
  )
  h1, xg0, hr0 = m0(h, W_emb, b_emb.reshape(1, F), Wg0, root0,
                    bias0.reshape(1, F))

  a0p0, a0p1 = _spmm_kernel(row, col, dis, p0, xg0)

  m1 = pl.pallas_call(
      _m1_body,
      grid=(GRID,),
      in_specs=[_row_spec(), _row_spec(), _row_spec(), _row_spec(),
                _full_spec((F, F)), _full_spec((F, F)), _full_spec((1, F))],
      out_specs=[_row_spec(), _row_spec(), _row_spec()],
      out_shape=[jax.ShapeDtypeStruct((N, F), jnp.float32)] * 3,
  )
  h2, xg1, hr1 = m1(h1, a0p0, a0p1, hr0, Wg1, root1, bias1.reshape(1, F))

  a1p0, a1p1 = _spmm_kernel(row, col, dis, p1, xg1)

  Wm_p = jnp.pad(Wm, ((0, 0), (0, F - Wm.shape[1])))
  bm_p = jnp.pad(bm, (0, F - bm.shape[0])).reshape(1, F)
  m2 = pl.pallas_call(
      _m2_body,
      grid=(GRID,),
      in_specs=[_row_spec(), _row_spec(), _row_spec(), _row_spec(),
                _full_spec((F, F)), _full_spec((1, F))],
      out_specs=_row_spec(),
      out_shape=jax.ShapeDtypeStruct((N, F), jnp.float32),
  )
  out = m2(h2, a1p0, a1p1, hr1, Wm_p, bm_p)
  return out[:, :Wm.shape[1]]


# speedup vs baseline: 12.0226x; 1.3154x over previous
import functools

import jax
import jax.numpy as jnp
from jax import lax
from jax.experimental import pallas as pl
from jax.experimental.pallas import tpu as pltpu
from jax.experimental.pallas import tpu_sc as plsc

N = 10000
E = 320000
F = 128
EPS = 1e-15

NC = 2
NS = 16
NW = NC * NS

N_PAD = 10240
RPT = N_PAD // NS
EPT = E // NW
CH = 80
NCHUNK = EPT // CH
DEG_LANES = 16

_MESH = dict(core_axis_name="c", subcore_axis_name="s", num_cores=NC,
             num_subcores=NS)

BLK = 2000
GRID = N // BLK


def _m0_body(h_ref, wemb_ref, bemb_ref, wg_ref, root_ref, bias_ref,
             h1_ref, xg_ref, hr_ref):
  h1 = jnp.dot(h_ref[...], wemb_ref[...],
               preferred_element_type=jnp.float32) + bemb_ref[...]
  h1_ref[...] = h1
  xg_ref[...] = jnp.dot(h1, wg_ref[...], preferred_element_type=jnp.float32)
  hr_ref[...] = jnp.dot(h1, root_ref[...],
                        preferred_element_type=jnp.float32) + bias_ref[...]


def _m1_body(h1_ref, ap0_ref, ap1_ref, hr_ref, wg_ref, root_ref, bias_ref,
             h2_ref, xg_ref, hr1_ref):
  aggr = ap0_ref[...] + ap1_ref[...]
  h2 = h1_ref[...] + jax.nn.relu(aggr + hr_ref[...])
  h2_ref[...] = h2
  xg_ref[...] = jnp.dot(h2, wg_ref[...], preferred_element_type=jnp.float32)
  hr1_ref[...] = jnp.dot(h2, root_ref[...],
                         preferred_element_type=jnp.float32) + bias_ref[...]


def _m2_body(h2_ref, ap0_ref, ap1_ref, hr_ref, wm_ref, bm_ref, o_ref):
  aggr = ap0_ref[...] + ap1_ref[...]
  h3 = h2_ref[...] + jax.nn.relu(aggr + hr_ref[...])
  o_ref[...] = jnp.dot(h3, wm_ref[...],
                       preferred_element_type=jnp.float32) + bm_ref[...]


def _row_spec():
  return pl.BlockSpec((BLK, F), lambda i: (i, 0))


def _full_spec(shape):
  nd = len(shape)
  return pl.BlockSpec(shape, lambda i: (0,) * nd)


def _part_spec():
  return pl.BlockSpec((NC, BLK, F), lambda i: (0, i, 0))


def _deg_body(row_hbm, ew_hbm, out0_hbm, out1_hbm, zbuf, row_v, ew_v, accum):
  cid = lax.axis_index("c")
  sid = lax.axis_index("s")
  wid = cid * NS + sid
  zero = jnp.zeros((16,), jnp.float32)

  def zrow(i, _):
    zbuf[pl.ds(i * 16, 16)] = zero
    return 0

  lax.fori_loop(0, RPT // 16, zrow, 0)
  pltpu.sync_copy(zbuf, accum.at[pl.ds(sid * RPT, RPT)])
  plsc.subcore_barrier()

  ebase = wid * EPT

  def chunk(c, _):
    base = ebase + c * CH
    pltpu.sync_copy(row_hbm.at[pl.ds(base, CH)], row_v.at[0])
    pltpu.sync_copy(ew_hbm.at[pl.ds(base, CH)], ew_v.at[0])
    pltpu.sync_copy(ew_v.at[0], accum.at[row_v.at[0]], add=True)
    return 0

  lax.fori_loop(0, NCHUNK, chunk, 0)
  plsc.subcore_barrier()

  @pl.when(cid == 0)
  def _():
    pltpu.sync_copy(accum.at[pl.ds(sid * RPT, RPT)],
                    out0_hbm.at[pl.ds(sid * RPT, RPT)])

  @pl.when(cid == 1)
  def _():
    pltpu.sync_copy(accum.at[pl.ds(sid * RPT, RPT)],
                    out1_hbm.at[pl.ds(sid * RPT, RPT)])


_NPW = N_PAD // NW


def _dis_body(deg0_hbm, deg1_hbm, dis_hbm, s0, s1, dv):
  cid = lax.axis_index("c")
  sid = lax.axis_index("s")
  wid = cid * NS + sid
  base = wid * _NPW
  pltpu.sync_copy(deg0_hbm.at[pl.ds(base, _NPW)], s0)
  pltpu.sync_copy(deg1_hbm.at[pl.ds(base, _NPW)], s1)
  half = jnp.full((16,), 0.5, jnp.float32)
  three_half = jnp.full((16,), 1.5, jnp.float32)
  magic = jnp.full((16,), 0x5F3759DF, jnp.int32)
  fzero = jnp.zeros((16,), jnp.float32)

  def grp(g, _):
    sl = pl.ds(g * 16, 16)
    d = s0[sl] + s1[sl]
    y = lax.bitcast_convert_type(
        magic - (lax.bitcast_convert_type(d, jnp.int32) >> 1), jnp.float32)
    hx = d * half
    y = y * (three_half - hx * y * y)
    y = y * (three_half - hx * y * y)
    y = y * (three_half - hx * y * y)
    dv[sl] = jnp.where(d > fzero, y, fzero)
    return 0

  lax.fori_loop(0, _NPW // 16, grp, 0)
  pltpu.sync_copy(dv, dis_hbm.at[pl.ds(base, _NPW)])


def _spmm_body(row_hbm, col_hbm, dis_hbm, p_hbm, xg_hbm,
               out0_hbm, out1_hbm,
               dis_s, p_v, row_v, col_v, dr_v, dc_v, gbuf, zbuf, accum,
               gsem0, gsem1, ssem0, ssem1):
  cid = lax.axis_index("c")
  sid = lax.axis_index("s")
  wid = cid * NS + sid
  zero = jnp.zeros((16,), jnp.float32)

  def zrow(i, _):
    for j in range(F // 16):
      zbuf[i, pl.ds(j * 16, 16)] = zero
    return 0

  lax.fori_loop(0, 128, zrow, 0)
  for t in range(RPT // 128):
    pltpu.sync_copy(zbuf, accum.at[pl.ds(sid * RPT + t * 128, 128)])

  @pl.when(sid == 0)
  def _():
    pltpu.sync_copy(dis_hbm, dis_s)

  pltpu.sync_copy(p_hbm, p_v)
  plsc.subcore_barrier()

  pvec = p_v[...]
  a0 = jnp.broadcast_to(lax.slice(pvec, (0,), (1,)), (16,))
  a1 = jnp.broadcast_to(lax.slice(pvec, (1,), (2,)), (16,))
  bb = jnp.broadcast_to(lax.slice(pvec, (2,), (3,)), (16,))
  mu = jnp.broadcast_to(lax.slice(pvec, (3,), (4,)), (16,))
  c2 = jnp.broadcast_to(lax.slice(pvec, (4,), (5,)), (16,))
  one = jnp.ones((16,), jnp.float32)
  two = jnp.full((16,), 2.0, jnp.float32)

  ebase = wid * EPT
  gsems = (gsem0, gsem1)
  ssems = (ssem0, ssem1)

  def stage_idx(c, p):
    base = ebase + c * CH
    pltpu.sync_copy(row_hbm.at[pl.ds(base, CH)], row_v.at[p])
    pltpu.sync_copy(col_hbm.at[pl.ds(base, CH)], col_v.at[p])
    pltpu.sync_copy(dis_s.at[row_v.at[p]], dr_v.at[p])
    pltpu.sync_copy(dis_s.at[col_v.at[p]], dc_v.at[p])

  def issue_gather(p):
    pltpu.async_copy(xg_hbm.at[row_v.at[p]], gbuf.at[p], gsems[p])

  def wait_gather(p):
    pltpu.make_async_copy(xg_hbm.at[row_v.at[p]], gbuf.at[p],
                          gsems[p]).wait()

  def wait_scatter(p):
    pltpu.make_async_copy(gbuf.at[p], accum.at[col_v.at[p]],
                          ssems[p]).wait()

  def compute(p):
    for g in range(CH // 16):
      x = a0 * dr_v[p, pl.ds(g * 16, 16)] + \
          a1 * dc_v[p, pl.ds(g * 16, 16)] + bb
      t = one - two / (jnp.exp(x + x) + one)
      d = t - mu
      w = jnp.exp(c2 * d * d)
      for i in range(16):
        wb = jnp.broadcast_to(lax.slice(w, (i,), (i + 1,)), (16,))
        r = g * 16 + i
        for j in range(F // 16):
          sl = pl.ds(j * 16, 16)
          gbuf[p, r, sl] = gbuf[p, r, sl] * wb

  def scatter(p):
    pltpu.async_copy(gbuf.at[p], accum.at[col_v.at[p]], ssems[p], add=True)

  stage_idx(0, 0)
  issue_gather(0)
  stage_idx(1, 1)
  issue_gather(1)
  wait_gather(0)
  compute(0)
  scatter(0)

  def chunk2(i, _):
    c = 2 * i + 1
    for p in (1, 0):
      q = 1 - p
      wait_scatter(q)
      stage_idx(c + 1, q)
      issue_gather(q)
      wait_gather(p)
      compute(p)
      scatter(p)
      c += 1
    return 0

  lax.fori_loop(0, (NCHUNK - 1) // 2, chunk2, 0)
  wait_scatter(0)
  wait_gather(1)
  plsc.subcore_barrier()

  @pl.when(cid == 0)
  def _():
    pltpu.sync_copy(accum.at[pl.ds(sid * RPT, RPT)],
                    out0_hbm.at[pl.ds(sid * RPT, RPT)])

  @pl.when(cid == 1)
  def _():
    pltpu.sync_copy(accum.at[pl.ds(sid * RPT, RPT)],
                    out1_hbm.at[pl.ds(sid * RPT, RPT)])


@functools.lru_cache(maxsize=1)
def _sc_kernels():
  mesh = plsc.VectorSubcoreMesh(**_MESH)
  deg_k = pl.kernel(
      _deg_body,
      out_type=[jax.ShapeDtypeStruct((N_PAD,), jnp.float32)] * 2,
      mesh=mesh,
      scratch_types=[
          pltpu.VMEM((RPT,), jnp.float32),
          pltpu.VMEM((2, CH), jnp.int32),
          pltpu.VMEM((2, CH), jnp.float32),
          pltpu.VMEM_SHARED((N_PAD,), jnp.float32),
      ],
  )
  dis_k = pl.kernel(
      _dis_body,
      out_type=jax.ShapeDtypeStruct((N_PAD,), jnp.float32),
      mesh=mesh,
      scratch_types=[
          pltpu.VMEM((_NPW,), jnp.float32),
          pltpu.VMEM((_NPW,), jnp.float32),
          pltpu.VMEM((_NPW,), jnp.float32),
      ],
  )
  spmm_k = pl.kernel(
      _spmm_body,
      out_type=[jax.ShapeDtypeStruct((N_PAD, F), jnp.float32)] * 2,
      mesh=mesh,
      scratch_types=[
          pltpu.VMEM_SHARED((N_PAD,), jnp.float32),
          pltpu.VMEM((16,), jnp.float32),
          pltpu.VMEM((2, CH), jnp.int32),
          pltpu.VMEM((2, CH), jnp.int32),
          pltpu.VMEM((2, CH), jnp.float32),
          pltpu.VMEM((2, CH), jnp.float32),
          pltpu.VMEM((2, CH, F), jnp.float32),
          pltpu.VMEM((128, F), jnp.float32),
          pltpu.VMEM_SHARED((N_PAD, F), jnp.float32),
          pltpu.SemaphoreType.DMA,
          pltpu.SemaphoreType.DMA,
          pltpu.SemaphoreType.DMA,
          pltpu.SemaphoreType.DMA,
      ],
  )
  return deg_k, dis_k, spmm_k


def _params_vec(Wp, bp, mu, sigma):
  a0 = Wp[0, 0]
  a1 = Wp[1, 0]
  b = bp[0]
  m = mu[0, 0]
  c2 = -0.5 / (EPS + sigma[0, 0] ** 2)
  return jnp.stack([a0, a1, b, m, c2,
                    0., 0., 0., 0., 0., 0., 0., 0., 0., 0., 0.]
                   ).astype(jnp.float32)


def kernel(h, edge_index, edge_weight, W_emb, b_emb, Wp0, bp0, Wg0, mu0,
           sigma0, root0, bias0, Wp1, bp1, Wg1, mu1, sigma1, root1, bias1,
           Wm, bm):
  row = jnp.pad(edge_index[0], (0, CH))
  col = jnp.pad(edge_index[1], (0, CH))
  ew = jnp.pad(edge_weight, (0, CH))
  _deg_kernel, _dis_kernel, _spmm_kernel = _sc_kernels()

  deg0, deg1 = _deg_kernel(row, ew)
  dis = _dis_kernel(deg0, deg1)

  p0 = _params_vec(Wp0, bp0, mu0, sigma0)
  p1 = _params_vec(Wp1, bp1, mu1, sigma1)

  m0 = pl.pallas_call(
      _m0_body,
      grid=(GRID,),
      in_specs=[_row_spec(), _full_spec((F, F)), _full_spec((1, F)),
                _full_spec((F, F)), _full_spec((F, F)), _full_spec((1, F))],
      out_specs=[_row_spec(), _row_spec(), _row_spec()],
      out_shape=[jax.ShapeDtypeStruct((N, F), jnp.float32)] * 3,
  )
  h1, xg0, hr0 = m0(h, W_emb, b_emb.reshape(1, F), Wg0, root0,
                    bias0.reshape(1, F))

  a0p0, a0p1 = _spmm_kernel(row, col, dis, p0, xg0)

  m1 = pl.pallas_call(
      _m1_body,
      grid=(GRID,),
      in_specs=[_row_spec(), _row_spec(), _row_spec(), _row_spec(),
                _full_spec((F, F)), _full_spec((F, F)), _full_spec((1, F))],
      out_specs=[_row_spec(), _row_spec(), _row_spec()],
      out_shape=[jax.ShapeDtypeStruct((N, F), jnp.float32)] * 3,
  )
  h2, xg1, hr1 = m1(h1, a0p0, a0p1, hr0, Wg1, root1, bias1.reshape(1, F))

  a1p0, a1p1 = _spmm_kernel(row, col, dis, p1, xg1)

  Wm_p = jnp.pad(Wm, ((0, 0), (0, F - Wm.shape[1])))
  bm_p = jnp.pad(bm, (0, F - bm.shape[0])).reshape(1, F)
  m2 = pl.pallas_call(
      _m2_body,
      grid=(GRID,),
      in_specs=[_row_spec(), _row_spec(), _row_spec(), _row_spec(),
                _full_spec((F, F)), _full_spec((1, F))],
      out_specs=_row_spec(),
      out_shape=jax.ShapeDtypeStruct((N, F), jnp.float32),
  )
  out = m2(h2, a1p0, a1p1, hr1, Wm_p, bm_p)
  return out[:, :Wm.shape[1]]


# speedup vs baseline: 14.0084x; 1.1652x over previous
import functools

import jax
import jax.numpy as jnp
from jax import lax
from jax.experimental import pallas as pl
from jax.experimental.pallas import tpu as pltpu
from jax.experimental.pallas import tpu_sc as plsc

N = 10000
E = 320000
F = 128
EPS = 1e-15

NC = 2
NS = 16
NW = NC * NS

N_PAD = 10240
RPT = N_PAD // NS
EPT = E // NW
CH = 80
NCHUNK = EPT // CH
DEG_LANES = 16

_MESH = dict(core_axis_name="c", subcore_axis_name="s", num_cores=NC,
             num_subcores=NS)

BLK = 2000
GRID = N // BLK


def _m0_body(h_ref, wemb_ref, bemb_ref, wg_ref, root_ref, bias_ref,
             h1_ref, xg_ref, hr_ref):
  h1 = jnp.dot(h_ref[...], wemb_ref[...],
               preferred_element_type=jnp.float32) + bemb_ref[...]
  h1_ref[...] = h1
  xg_ref[...] = jnp.dot(h1, wg_ref[...], preferred_element_type=jnp.float32)
  hr_ref[...] = jnp.dot(h1, root_ref[...],
                        preferred_element_type=jnp.float32) + bias_ref[...]


def _m1_body(h1_ref, ap0_ref, ap1_ref, hr_ref, wg_ref, root_ref, bias_ref,
             h2_ref, xg_ref, hr1_ref):
  aggr = ap0_ref[...] + ap1_ref[...]
  h2 = h1_ref[...] + jax.nn.relu(aggr + hr_ref[...])
  h2_ref[...] = h2
  xg_ref[...] = jnp.dot(h2, wg_ref[...], preferred_element_type=jnp.float32)
  hr1_ref[...] = jnp.dot(h2, root_ref[...],
                         preferred_element_type=jnp.float32) + bias_ref[...]


def _m2_body(h2_ref, ap0_ref, ap1_ref, hr_ref, wm_ref, bm_ref, o_ref):
  aggr = ap0_ref[...] + ap1_ref[...]
  h3 = h2_ref[...] + jax.nn.relu(aggr + hr_ref[...])
  o_ref[...] = jnp.dot(h3, wm_ref[...],
                       preferred_element_type=jnp.float32) + bm_ref[...]


def _row_spec():
  return pl.BlockSpec((BLK, F), lambda i: (i, 0))


def _full_spec(shape):
  nd = len(shape)
  return pl.BlockSpec(shape, lambda i: (0,) * nd)


def _part_spec():
  return pl.BlockSpec((NC, BLK, F), lambda i: (0, i, 0))


def _deg_body(row_hbm, ew_hbm, out0_hbm, out1_hbm, zbuf, row_v, ew_v,
              accum):
  cid = lax.axis_index("c")
  sid = lax.axis_index("s")
  wid = cid * NS + sid
  zero = jnp.zeros((16,), jnp.float32)

  def zrow(i, _):
    zbuf[pl.ds(i * 16, 16)] = zero
    return 0

  lax.fori_loop(0, RPT // 16, zrow, 0)
  pltpu.sync_copy(zbuf, accum.at[pl.ds(sid * RPT, RPT)])
  plsc.subcore_barrier()

  ebase = wid * EPT

  def chunk(c, _):
    base = ebase + c * CH
    pltpu.sync_copy(row_hbm.at[pl.ds(base, CH)], row_v.at[0])
    pltpu.sync_copy(ew_hbm.at[pl.ds(base, CH)], ew_v.at[0])
    pltpu.sync_copy(ew_v.at[0], accum.at[row_v.at[0]], add=True)
    return 0

  lax.fori_loop(0, NCHUNK, chunk, 0)
  plsc.subcore_barrier()

  @pl.when(cid == 0)
  def _():
    pltpu.sync_copy(accum.at[pl.ds(sid * RPT, RPT)],
                    out0_hbm.at[pl.ds(sid * RPT, RPT)])

  @pl.when(cid == 1)
  def _():
    pltpu.sync_copy(accum.at[pl.ds(sid * RPT, RPT)],
                    out1_hbm.at[pl.ds(sid * RPT, RPT)])


_NPW = N_PAD // NW


def _dis_body(deg0_hbm, deg1_hbm, dis_hbm, s0, s1, dv):
  cid = lax.axis_index("c")
  sid = lax.axis_index("s")
  wid = cid * NS + sid
  base = wid * _NPW
  pltpu.sync_copy(deg0_hbm.at[pl.ds(base, _NPW)], s0)
  pltpu.sync_copy(deg1_hbm.at[pl.ds(base, _NPW)], s1)
  half = jnp.full((16,), 0.5, jnp.float32)
  three_half = jnp.full((16,), 1.5, jnp.float32)
  magic = jnp.full((16,), 0x5F3759DF, jnp.int32)
  fzero = jnp.zeros((16,), jnp.float32)

  def grp(g, _):
    sl = pl.ds(g * 16, 16)
    d = s0[sl] + s1[sl]
    y = lax.bitcast_convert_type(
        magic - (lax.bitcast_convert_type(d, jnp.int32) >> 1), jnp.float32)
    hx = d * half
    y = y * (three_half - hx * y * y)
    y = y * (three_half - hx * y * y)
    y = y * (three_half - hx * y * y)
    dv[sl] = jnp.where(d > fzero, y, fzero)
    return 0

  lax.fori_loop(0, _NPW // 16, grp, 0)
  pltpu.sync_copy(dv, dis_hbm.at[pl.ds(base, _NPW)])


def _spmm_body(row_hbm, col_hbm, dis_hbm, p_hbm, xg_hbm,
               out0_hbm, out1_hbm,
               dis_s, p_v, row_v, col_v, scol_v, dr_v, dc_v, gbuf, zbuf,
               accum, gsem0, gsem1, ssem0, ssem1):
  cid = lax.axis_index("c")
  sid = lax.axis_index("s")
  wid = cid * NS + sid
  zero = jnp.zeros((16,), jnp.float32)

  def zrow(i, _):
    for j in range(F // 16):
      zbuf[i, pl.ds(j * 16, 16)] = zero
    return 0

  lax.fori_loop(0, 128, zrow, 0)
  for t in range(RPT // 128):
    pltpu.sync_copy(zbuf, accum.at[pl.ds(sid * RPT + t * 128, 128)])

  @pl.when(sid == 0)
  def _():
    pltpu.sync_copy(dis_hbm, dis_s)

  pltpu.sync_copy(p_hbm, p_v)
  plsc.subcore_barrier()

  pvec = p_v[...]
  a0 = jnp.broadcast_to(lax.slice(pvec, (0,), (1,)), (16,))
  a1 = jnp.broadcast_to(lax.slice(pvec, (1,), (2,)), (16,))
  bb = jnp.broadcast_to(lax.slice(pvec, (2,), (3,)), (16,))
  mu = jnp.broadcast_to(lax.slice(pvec, (3,), (4,)), (16,))
  c2 = jnp.broadcast_to(lax.slice(pvec, (4,), (5,)), (16,))
  one = jnp.ones((16,), jnp.float32)
  two = jnp.full((16,), 2.0, jnp.float32)

  ebase = wid * EPT
  gsems = (gsem0, gsem1)
  ssems = (ssem0, ssem1)

  def load_idx(c, p):
    base = ebase + c * CH
    pltpu.sync_copy(row_hbm.at[pl.ds(base, CH)], row_v.at[p])
    pltpu.sync_copy(col_hbm.at[pl.ds(base, CH)], col_v.at[p])

  def issue_gather(p):
    pltpu.async_copy(xg_hbm.at[row_v.at[p]], gbuf.at[p], gsems[p])

  def wait_gather(p):
    pltpu.make_async_copy(xg_hbm.at[row_v.at[p]], gbuf.at[p],
                          gsems[p]).wait()

  def load_dis(p):
    pltpu.sync_copy(dis_s.at[row_v.at[p]], dr_v.at[p])
    pltpu.sync_copy(dis_s.at[col_v.at[p]], dc_v.at[p])

  def stash_col(p):
    for g in range(CH // 16):
      sl = pl.ds(g * 16, 16)
      scol_v[p, sl] = col_v[p, sl]

  def scatter(p):
    pltpu.async_copy(gbuf.at[p], accum.at[scol_v.at[p]], ssems[p], add=True)

  def wait_scatter(p):
    pltpu.make_async_copy(gbuf.at[p], accum.at[scol_v.at[p]],
                          ssems[p]).wait()

  def compute(p):
    for g in range(CH // 16):
      x = a0 * dr_v[p, pl.ds(g * 16, 16)] + \
          a1 * dc_v[p, pl.ds(g * 16, 16)] + bb
      t = one - two / (jnp.exp(x + x) + one)
      d = t - mu
      w = jnp.exp(c2 * d * d)
      for i in range(16):
        wb = jnp.broadcast_to(lax.slice(w, (i,), (i + 1,)), (16,))
        r = g * 16 + i
        for j in range(F // 16):
          sl = pl.ds(j * 16, 16)
          gbuf[p, r, sl] = gbuf[p, r, sl] * wb

  def phase(c, s, first):
    t = 1 - s
    load_idx(c + 1, t)
    load_dis(t)
    wait_gather(s)
    stash_col(s)
    if not first:
      wait_scatter(t)
    issue_gather(t)
    compute(s)
    scatter(s)

  load_idx(0, 0)
  load_dis(0)
  issue_gather(0)
  phase(0, 0, True)

  def chunk2(i, _):
    c = 2 * i + 1
    phase(c, 1, False)
    phase(c + 1, 0, False)
    return 0

  lax.fori_loop(0, (NCHUNK - 1) // 2, chunk2, 0)
  wait_scatter(0)
  wait_gather(1)
  plsc.subcore_barrier()

  @pl.when(cid == 0)
  def _():
    pltpu.sync_copy(accum.at[pl.ds(sid * RPT, RPT)],
                    out0_hbm.at[pl.ds(sid * RPT, RPT)])

  @pl.when(cid == 1)
  def _():
    pltpu.sync_copy(accum.at[pl.ds(sid * RPT, RPT)],
                    out1_hbm.at[pl.ds(sid * RPT, RPT)])


@functools.lru_cache(maxsize=1)
def _sc_kernels():
  mesh = plsc.VectorSubcoreMesh(**_MESH)
  deg_k = pl.kernel(
      _deg_body,
      out_type=[jax.ShapeDtypeStruct((N_PAD,), jnp.float32)] * 2,
      mesh=mesh,
      scratch_types=[
          pltpu.VMEM((RPT,), jnp.float32),
          pltpu.VMEM((2, CH), jnp.int32),
          pltpu.VMEM((2, CH), jnp.float32),
          pltpu.VMEM_SHARED((N_PAD,), jnp.float32),
      ],
  )
  dis_k = pl.kernel(
      _dis_body,
      out_type=jax.ShapeDtypeStruct((N_PAD,), jnp.float32),
      mesh=mesh,
      scratch_types=[
          pltpu.VMEM((_NPW,), jnp.float32),
          pltpu.VMEM((_NPW,), jnp.float32),
          pltpu.VMEM((_NPW,), jnp.float32),
      ],
  )
  spmm_k = pl.kernel(
      _spmm_body,
      out_type=[jax.ShapeDtypeStruct((N_PAD, F), jnp.float32)] * 2,
      mesh=mesh,
      scratch_types=[
          pltpu.VMEM_SHARED((N_PAD,), jnp.float32),
          pltpu.VMEM((16,), jnp.float32),
          pltpu.VMEM((2, CH), jnp.int32),
          pltpu.VMEM((2, CH), jnp.int32),
          pltpu.VMEM((2, CH), jnp.int32),
          pltpu.VMEM((2, CH), jnp.float32),
          pltpu.VMEM((2, CH), jnp.float32),
          pltpu.VMEM((2, CH, F), jnp.float32),
          pltpu.VMEM((128, F), jnp.float32),
          pltpu.VMEM_SHARED((N_PAD, F), jnp.float32),
          pltpu.SemaphoreType.DMA,
          pltpu.SemaphoreType.DMA,
          pltpu.SemaphoreType.DMA,
          pltpu.SemaphoreType.DMA,
      ],
  )
  return deg_k, dis_k, spmm_k


def _params_vec(Wp, bp, mu, sigma):
  a0 = Wp[0, 0]
  a1 = Wp[1, 0]
  b = bp[0]
  m = mu[0, 0]
  c2 = -0.5 / (EPS + sigma[0, 0] ** 2)
  return jnp.stack([a0, a1, b, m, c2,
                    0., 0., 0., 0., 0., 0., 0., 0., 0., 0., 0.]
                   ).astype(jnp.float32)


def kernel(h, edge_index, edge_weight, W_emb, b_emb, Wp0, bp0, Wg0, mu0,
           sigma0, root0, bias0, Wp1, bp1, Wg1, mu1, sigma1, root1, bias1,
           Wm, bm):
  row = jnp.pad(edge_index[0], (0, 2 * CH))
  col = jnp.pad(edge_index[1], (0, 2 * CH))
  ew = jnp.pad(edge_weight, (0, 2 * CH))
  _deg_kernel, _dis_kernel, _spmm_kernel = _sc_kernels()

  deg0, deg1 = _deg_kernel(row, ew)
  dis = _dis_kernel(deg0, deg1)

  p0 = _params_vec(Wp0, bp0, mu0, sigma0)
  p1 = _params_vec(Wp1, bp1, mu1, sigma1)

  m0 = pl.pallas_call(
      _m0_body,
      grid=(GRID,),
      in_specs=[_row_spec(), _full_spec((F, F)), _full_spec((1, F)),
                _full_spec((F, F)), _full_spec((F, F)), _full_spec((1, F))],
      out_specs=[_row_spec(), _row_spec(), _row_spec()],
      out_shape=[jax.ShapeDtypeStruct((N, F), jnp.float32)] * 3,
  )
  h1, xg0, hr0 = m0(h, W_emb, b_emb.reshape(1, F), Wg0, root0,
                    bias0.reshape(1, F))

  a0p0, a0p1 = _spmm_kernel(row, col, dis, p0, xg0)

  m1 = pl.pallas_call(
      _m1_body,
      grid=(GRID,),
      in_specs=[_row_spec(), _row_spec(), _row_spec(), _row_spec(),
                _full_spec((F, F)), _full_spec((F, F)), _full_spec((1, F))],
      out_specs=[_row_spec(), _row_spec(), _row_spec()],
      out_shape=[jax.ShapeDtypeStruct((N, F), jnp.float32)] * 3,
  )
  h2, xg1, hr1 = m1(h1, a0p0, a0p1, hr0, Wg1, root1, bias1.reshape(1, F))

  a1p0, a1p1 = _spmm_kernel(row, col, dis, p1, xg1)

  Wm_p = jnp.pad(Wm, ((0, 0), (0, F - Wm.shape[1])))
  bm_p = jnp.pad(bm, (0, F - bm.shape[0])).reshape(1, F)
  m2 = pl.pallas_call(
      _m2_body,
      grid=(GRID,),
      in_specs=[_row_spec(), _row_spec(), _row_spec(), _row_spec(),
                _full_spec((F, F)), _full_spec((1, F))],
      out_specs=_row_spec(),
      out_shape=jax.ShapeDtypeStruct((N, F), jnp.float32),
  )
  out = m2(h2, a1p0, a1p1, hr1, Wm_p, bm_p)
  return out[:, :Wm.shape[1]]


# speedup vs baseline: 14.2301x; 1.0158x over previous
import functools

import jax
import jax.numpy as jnp
from jax import lax
from jax.experimental import pallas as pl
from jax.experimental.pallas import tpu as pltpu
from jax.experimental.pallas import tpu_sc as plsc

N = 10000
E = 320000
F = 128
EPS = 1e-15

NC = 2
NS = 16
NW = NC * NS

N_PAD = 10240
RPT = N_PAD // NS
EPT = E // NW
CH = 80
NCHUNK = EPT // CH
DEG_LANES = 16

_MESH = dict(core_axis_name="c", subcore_axis_name="s", num_cores=NC,
             num_subcores=NS)

BLK = 2000
GRID = N // BLK


def _m0_body(h_ref, wemb_ref, bemb_ref, wg_ref, root_ref, bias_ref,
             h1_ref, xg_ref, hr_ref):
  h1 = jnp.dot(h_ref[...], wemb_ref[...],
               preferred_element_type=jnp.float32) + bemb_ref[...]
  h1_ref[...] = h1
  xg_ref[...] = jnp.dot(h1, wg_ref[...], preferred_element_type=jnp.float32)
  hr_ref[...] = jnp.dot(h1, root_ref[...],
                        preferred_element_type=jnp.float32) + bias_ref[...]


def _m1_body(h1_ref, ap0_ref, ap1_ref, hr_ref, wg_ref, root_ref, bias_ref,
             h2_ref, xg_ref, hr1_ref):
  aggr = ap0_ref[...] + ap1_ref[...]
  h2 = h1_ref[...] + jax.nn.relu(aggr + hr_ref[...])
  h2_ref[...] = h2
  xg_ref[...] = jnp.dot(h2, wg_ref[...], preferred_element_type=jnp.float32)
  hr1_ref[...] = jnp.dot(h2, root_ref[...],
                         preferred_element_type=jnp.float32) + bias_ref[...]


def _m2_body(h2_ref, ap0_ref, ap1_ref, hr_ref, wm_ref, bm_ref, o_ref):
  aggr = ap0_ref[...] + ap1_ref[...]
  h3 = h2_ref[...] + jax.nn.relu(aggr + hr_ref[...])
  o_ref[...] = jnp.dot(h3, wm_ref[...],
                       preferred_element_type=jnp.float32) + bm_ref[...]


def _row_spec():
  return pl.BlockSpec((BLK, F), lambda i: (i, 0))


def _full_spec(shape):
  nd = len(shape)
  return pl.BlockSpec(shape, lambda i: (0,) * nd)


def _part_spec():
  return pl.BlockSpec((NC, BLK, F), lambda i: (0, i, 0))


def _deg_body(row_hbm, ew_hbm, out0_hbm, out1_hbm, zbuf, row_v, ew_v,
              srow_v, sew_v, accum, ssem0, ssem1):
  cid = lax.axis_index("c")
  sid = lax.axis_index("s")
  wid = cid * NS + sid
  zero = jnp.zeros((16,), jnp.float32)

  def zrow(i, _):
    zbuf[pl.ds(i * 16, 16)] = zero
    return 0

  lax.fori_loop(0, RPT // 16, zrow, 0)
  pltpu.sync_copy(zbuf, accum.at[pl.ds(sid * RPT, RPT)])
  plsc.subcore_barrier()

  ebase = wid * EPT
  ssems = (ssem0, ssem1)

  def load_idx(c, p):
    base = ebase + c * CH
    pltpu.sync_copy(row_hbm.at[pl.ds(base, CH)], row_v.at[p])
    pltpu.sync_copy(ew_hbm.at[pl.ds(base, CH)], ew_v.at[p])

  def stash(p):
    for g in range(CH // 16):
      sl = pl.ds(g * 16, 16)
      srow_v[p, sl] = row_v[p, sl]
      sew_v[p, sl] = ew_v[p, sl]

  def scatter(p):
    pltpu.async_copy(sew_v.at[p], accum.at[srow_v.at[p]], ssems[p],
                     add=True)

  def wait_scatter(p):
    pltpu.make_async_copy(sew_v.at[p], accum.at[srow_v.at[p]],
                          ssems[p]).wait()

  def phase(c, s, first):
    t = 1 - s
    load_idx(c + 1, t)
    stash(s)
    if not first:
      wait_scatter(t)
    scatter(s)

  load_idx(0, 0)
  phase(0, 0, True)

  def chunk2(i, _):
    c = 2 * i + 1
    phase(c, 1, False)
    phase(c + 1, 0, False)
    return 0

  lax.fori_loop(0, (NCHUNK - 1) // 2, chunk2, 0)
  wait_scatter(0)
  plsc.subcore_barrier()

  @pl.when(cid == 0)
  def _():
    pltpu.sync_copy(accum.at[pl.ds(sid * RPT, RPT)],
                    out0_hbm.at[pl.ds(sid * RPT, RPT)])

  @pl.when(cid == 1)
  def _():
    pltpu.sync_copy(accum.at[pl.ds(sid * RPT, RPT)],
                    out1_hbm.at[pl.ds(sid * RPT, RPT)])


_NPW = N_PAD // NW


def _dis_body(deg0_hbm, deg1_hbm, dis_hbm, s0, s1, dv):
  cid = lax.axis_index("c")
  sid = lax.axis_index("s")
  wid = cid * NS + sid
  base = wid * _NPW
  pltpu.sync_copy(deg0_hbm.at[pl.ds(base, _NPW)], s0)
  pltpu.sync_copy(deg1_hbm.at[pl.ds(base, _NPW)], s1)
  half = jnp.full((16,), 0.5, jnp.float32)
  three_half = jnp.full((16,), 1.5, jnp.float32)
  magic = jnp.full((16,), 0x5F3759DF, jnp.int32)
  fzero = jnp.zeros((16,), jnp.float32)

  def grp(g, _):
    sl = pl.ds(g * 16, 16)
    d = s0[sl] + s1[sl]
    y = lax.bitcast_convert_type(
        magic - (lax.bitcast_convert_type(d, jnp.int32) >> 1), jnp.float32)
    hx = d * half
    y = y * (three_half - hx * y * y)
    y = y * (three_half - hx * y * y)
    y = y * (three_half - hx * y * y)
    dv[sl] = jnp.where(d > fzero, y, fzero)
    return 0

  lax.fori_loop(0, _NPW // 16, grp, 0)
  pltpu.sync_copy(dv, dis_hbm.at[pl.ds(base, _NPW)])


def _spmm_body(row_hbm, col_hbm, dis_hbm, p_hbm, xg_hbm,
               out0_hbm, out1_hbm,
               dis_s, p_v, row_v, col_v, scol_v, dr_v, dc_v, gbuf, zbuf,
               accum, gsem0, gsem1, ssem0, ssem1):
  cid = lax.axis_index("c")
  sid = lax.axis_index("s")
  wid = cid * NS + sid
  zero = jnp.zeros((16,), jnp.float32)

  def zrow(i, _):
    for j in range(F // 16):
      zbuf[i, pl.ds(j * 16, 16)] = zero
    return 0

  lax.fori_loop(0, 128, zrow, 0)
  for t in range(RPT // 128):
    pltpu.sync_copy(zbuf, accum.at[pl.ds(sid * RPT + t * 128, 128)])

  @pl.when(sid == 0)
  def _():
    pltpu.sync_copy(dis_hbm, dis_s)

  pltpu.sync_copy(p_hbm, p_v)
  plsc.subcore_barrier()

  pvec = p_v[...]
  a0 = jnp.broadcast_to(lax.slice(pvec, (0,), (1,)), (16,))
  a1 = jnp.broadcast_to(lax.slice(pvec, (1,), (2,)), (16,))
  bb = jnp.broadcast_to(lax.slice(pvec, (2,), (3,)), (16,))
  mu = jnp.broadcast_to(lax.slice(pvec, (3,), (4,)), (16,))
  c2 = jnp.broadcast_to(lax.slice(pvec, (4,), (5,)), (16,))
  one = jnp.ones((16,), jnp.float32)
  two = jnp.full((16,), 2.0, jnp.float32)

  ebase = wid * EPT
  gsems = (gsem0, gsem1)
  ssems = (ssem0, ssem1)

  def load_idx(c, p):
    base = ebase + c * CH
    pltpu.sync_copy(row_hbm.at[pl.ds(base, CH)], row_v.at[p])
    pltpu.sync_copy(col_hbm.at[pl.ds(base, CH)], col_v.at[p])

  def issue_gather(p):
    pltpu.async_copy(xg_hbm.at[row_v.at[p]], gbuf.at[p], gsems[p])

  def wait_gather(p):
    pltpu.make_async_copy(xg_hbm.at[row_v.at[p]], gbuf.at[p],
                          gsems[p]).wait()

  def load_dis(p):
    pltpu.sync_copy(dis_s.at[row_v.at[p]], dr_v.at[p])
    pltpu.sync_copy(dis_s.at[col_v.at[p]], dc_v.at[p])

  def stash_col(p):
    for g in range(CH // 16):
      sl = pl.ds(g * 16, 16)
      scol_v[p, sl] = col_v[p, sl]

  def scatter(p):
    pltpu.async_copy(gbuf.at[p], accum.at[scol_v.at[p]], ssems[p], add=True)

  def wait_scatter(p):
    pltpu.make_async_copy(gbuf.at[p], accum.at[scol_v.at[p]],
                          ssems[p]).wait()

  def compute(p):
    for g in range(CH // 16):
      x = a0 * dr_v[p, pl.ds(g * 16, 16)] + \
          a1 * dc_v[p, pl.ds(g * 16, 16)] + bb
      t = one - two / (jnp.exp(x + x) + one)
      d = t - mu
      w = jnp.exp(c2 * d * d)
      for i in range(16):
        wb = jnp.broadcast_to(lax.slice(w, (i,), (i + 1,)), (16,))
        r = g * 16 + i
        for j in range(F // 16):
          sl = pl.ds(j * 16, 16)
          gbuf[p, r, sl] = gbuf[p, r, sl] * wb

  def phase(c, s, first):
    t = 1 - s
    load_idx(c + 1, t)
    load_dis(t)
    wait_gather(s)
    stash_col(s)
    if not first:
      wait_scatter(t)
    issue_gather(t)
    compute(s)
    scatter(s)

  load_idx(0, 0)
  load_dis(0)
  issue_gather(0)
  phase(0, 0, True)

  def chunk2(i, _):
    c = 2 * i + 1
    phase(c, 1, False)
    phase(c + 1, 0, False)
    return 0

  lax.fori_loop(0, (NCHUNK - 1) // 2, chunk2, 0)
  wait_scatter(0)
  wait_gather(1)
  plsc.subcore_barrier()

  @pl.when(cid == 0)
  def _():
    pltpu.sync_copy(accum.at[pl.ds(sid * RPT, RPT)],
                    out0_hbm.at[pl.ds(sid * RPT, RPT)])

  @pl.when(cid == 1)
  def _():
    pltpu.sync_copy(accum.at[pl.ds(sid * RPT, RPT)],
                    out1_hbm.at[pl.ds(sid * RPT, RPT)])


@functools.lru_cache(maxsize=1)
def _sc_kernels():
  mesh = plsc.VectorSubcoreMesh(**_MESH)
  deg_k = pl.kernel(
      _deg_body,
      out_type=[jax.ShapeDtypeStruct((N_PAD,), jnp.float32)] * 2,
      mesh=mesh,
      scratch_types=[
          pltpu.VMEM((RPT,), jnp.float32),
          pltpu.VMEM((2, CH), jnp.int32),
          pltpu.VMEM((2, CH), jnp.float32),
          pltpu.VMEM((2, CH), jnp.int32),
          pltpu.VMEM((2, CH), jnp.float32),
          pltpu.VMEM_SHARED((N_PAD,), jnp.float32),
          pltpu.SemaphoreType.DMA,
          pltpu.SemaphoreType.DMA,
      ],
  )
  dis_k = pl.kernel(
      _dis_body,
      out_type=jax.ShapeDtypeStruct((N_PAD,), jnp.float32),
      mesh=mesh,
      scratch_types=[
          pltpu.VMEM((_NPW,), jnp.float32),
          pltpu.VMEM((_NPW,), jnp.float32),
          pltpu.VMEM((_NPW,), jnp.float32),
      ],
  )
  spmm_k = pl.kernel(
      _spmm_body,
      out_type=[jax.ShapeDtypeStruct((N_PAD, F), jnp.float32)] * 2,
      mesh=mesh,
      scratch_types=[
          pltpu.VMEM_SHARED((N_PAD,), jnp.float32),
          pltpu.VMEM((16,), jnp.float32),
          pltpu.VMEM((2, CH), jnp.int32),
          pltpu.VMEM((2, CH), jnp.int32),
          pltpu.VMEM((2, CH), jnp.int32),
          pltpu.VMEM((2, CH), jnp.float32),
          pltpu.VMEM((2, CH), jnp.float32),
          pltpu.VMEM((2, CH, F), jnp.float32),
          pltpu.VMEM((128, F), jnp.float32),
          pltpu.VMEM_SHARED((N_PAD, F), jnp.float32),
          pltpu.SemaphoreType.DMA,
          pltpu.SemaphoreType.DMA,
          pltpu.SemaphoreType.DMA,
          pltpu.SemaphoreType.DMA,
      ],
  )
  return deg_k, dis_k, spmm_k


def _params_vec(Wp, bp, mu, sigma):
  a0 = Wp[0, 0]
  a1 = Wp[1, 0]
  b = bp[0]
  m = mu[0, 0]
  c2 = -0.5 / (EPS + sigma[0, 0] ** 2)
  return jnp.stack([a0, a1, b, m, c2,
                    0., 0., 0., 0., 0., 0., 0., 0., 0., 0., 0.]
                   ).astype(jnp.float32)


def kernel(h, edge_index, edge_weight, W_emb, b_emb, Wp0, bp0, Wg0, mu0,
           sigma0, root0, bias0, Wp1, bp1, Wg1, mu1, sigma1, root1, bias1,
           Wm, bm):
  row = jnp.pad(edge_index[0], (0, 2 * CH))
  col = jnp.pad(edge_index[1], (0, 2 * CH))
  ew = jnp.pad(edge_weight, (0, 2 * CH))
  _deg_kernel, _dis_kernel, _spmm_kernel = _sc_kernels()

  deg0, deg1 = _deg_kernel(row, ew)
  dis = _dis_kernel(deg0, deg1)

  p0 = _params_vec(Wp0, bp0, mu0, sigma0)
  p1 = _params_vec(Wp1, bp1, mu1, sigma1)

  m0 = pl.pallas_call(
      _m0_body,
      grid=(GRID,),
      in_specs=[_row_spec(), _full_spec((F, F)), _full_spec((1, F)),
                _full_spec((F, F)), _full_spec((F, F)), _full_spec((1, F))],
      out_specs=[_row_spec(), _row_spec(), _row_spec()],
      out_shape=[jax.ShapeDtypeStruct((N, F), jnp.float32)] * 3,
  )
  h1, xg0, hr0 = m0(h, W_emb, b_emb.reshape(1, F), Wg0, root0,
                    bias0.reshape(1, F))

  a0p0, a0p1 = _spmm_kernel(row, col, dis, p0, xg0)

  m1 = pl.pallas_call(
      _m1_body,
      grid=(GRID,),
      in_specs=[_row_spec(), _row_spec(), _row_spec(), _row_spec(),
                _full_spec((F, F)), _full_spec((F, F)), _full_spec((1, F))],
      out_specs=[_row_spec(), _row_spec(), _row_spec()],
      out_shape=[jax.ShapeDtypeStruct((N, F), jnp.float32)] * 3,
  )
  h2, xg1, hr1 = m1(h1, a0p0, a0p1, hr0, Wg1, root1, bias1.reshape(1, F))

  a1p0, a1p1 = _spmm_kernel(row, col, dis, p1, xg1)

  Wm_p = jnp.pad(Wm, ((0, 0), (0, F - Wm.shape[1])))
  bm_p = jnp.pad(bm, (0, F - bm.shape[0])).reshape(1, F)
  m2 = pl.pallas_call(
      _m2_body,
      grid=(GRID,),
      in_specs=[_row_spec(), _row_spec(), _row_spec(), _row_spec(),
                _full_spec((F, F)), _full_spec((1, F))],
      out_specs=_row_spec(),
      out_shape=jax.ShapeDtypeStruct((N, F), jnp.float32),
  )
  out = m2(h2, a1p0, a1p1, hr1, Wm_p, bm_p)
  return out[:, :Wm.shape[1]]


# speedup vs baseline: 16.2959x; 1.1452x over previous
import functools

import jax
import jax.numpy as jnp
from jax import lax
from jax.experimental import pallas as pl
from jax.experimental.pallas import tpu as pltpu
from jax.experimental.pallas import tpu_sc as plsc

N = 10000
E = 320000
F = 128
EPS = 1e-15

NC = 2
NS = 16
NW = NC * NS

N_PAD = 10240
RPT = N_PAD // NS
EPT = E // NW
CH = 80
NCHUNK = EPT // CH
DEG_LANES = 16

_MESH = dict(core_axis_name="c", subcore_axis_name="s", num_cores=NC,
             num_subcores=NS)

BLK = 2000
GRID = N // BLK


def _m0_body(h_ref, wemb_ref, bemb_ref, wg_ref, root_ref, bias_ref,
             h1_ref, xg_ref, hr_ref):
  h1 = jnp.dot(h_ref[...], wemb_ref[...],
               preferred_element_type=jnp.float32) + bemb_ref[...]
  h1_ref[...] = h1
  xg_ref[...] = jnp.dot(h1, wg_ref[...], preferred_element_type=jnp.float32)
  hr_ref[...] = jnp.dot(h1, root_ref[...],
                        preferred_element_type=jnp.float32) + bias_ref[...]


def _m1_body(h1_ref, ap0_ref, ap1_ref, hr_ref, wg_ref, root_ref, bias_ref,
             h2_ref, xg_ref, hr1_ref):
  aggr = ap0_ref[...] + ap1_ref[...]
  h2 = h1_ref[...] + jax.nn.relu(aggr + hr_ref[...])
  h2_ref[...] = h2
  xg_ref[...] = jnp.dot(h2, wg_ref[...], preferred_element_type=jnp.float32)
  hr1_ref[...] = jnp.dot(h2, root_ref[...],
                         preferred_element_type=jnp.float32) + bias_ref[...]


def _m2_body(h2_ref, ap0_ref, ap1_ref, hr_ref, wm_ref, bm_ref, o_ref):
  aggr = ap0_ref[...] + ap1_ref[...]
  h3 = h2_ref[...] + jax.nn.relu(aggr + hr_ref[...])
  o_ref[...] = jnp.dot(h3, wm_ref[...],
                       preferred_element_type=jnp.float32) + bm_ref[...]


def _row_spec():
  return pl.BlockSpec((BLK, F), lambda i: (i, 0))


def _full_spec(shape):
  nd = len(shape)
  return pl.BlockSpec(shape, lambda i: (0,) * nd)


def _part_spec():
  return pl.BlockSpec((NC, BLK, F), lambda i: (0, i, 0))


def _deg_body(row_hbm, ew_hbm, out0_hbm, out1_hbm, zbuf, row_v, ew_v,
              srow_v, sew_v, accum, rsem0, rsem1, esem0, esem1,
              ssem0, ssem1):
  cid = lax.axis_index("c")
  sid = lax.axis_index("s")
  wid = cid * NS + sid
  zero = jnp.zeros((16,), jnp.float32)

  def zrow(i, _):
    zbuf[pl.ds(i * 16, 16)] = zero
    return 0

  lax.fori_loop(0, RPT // 16, zrow, 0)
  pltpu.sync_copy(zbuf, accum.at[pl.ds(sid * RPT, RPT)])
  plsc.subcore_barrier()

  ebase = wid * EPT
  rsems = (rsem0, rsem1)
  esems = (esem0, esem1)
  ssems = (ssem0, ssem1)

  def issue_loads(c, p):
    base = ebase + c * CH
    pltpu.async_copy(row_hbm.at[pl.ds(base, CH)], row_v.at[p], rsems[p])
    pltpu.async_copy(ew_hbm.at[pl.ds(base, CH)], ew_v.at[p], esems[p])

  def wait_loads(c, p):
    base = ebase + c * CH
    pltpu.make_async_copy(row_hbm.at[pl.ds(base, CH)], row_v.at[p],
                          rsems[p]).wait()
    pltpu.make_async_copy(ew_hbm.at[pl.ds(base, CH)], ew_v.at[p],
                          esems[p]).wait()

  def stash(p):
    for g in range(CH // 16):
      sl = pl.ds(g * 16, 16)
      srow_v[p, sl] = row_v[p, sl]
      sew_v[p, sl] = ew_v[p, sl]

  def scatter(p):
    pltpu.async_copy(sew_v.at[p], accum.at[srow_v.at[p]], ssems[p],
                     add=True)

  def wait_scatter(p):
    pltpu.make_async_copy(sew_v.at[p], accum.at[srow_v.at[p]],
                          ssems[p]).wait()

  def phase(c, s, first):
    t = 1 - s
    wait_loads(c, s)
    stash(s)
    issue_loads(c + 2, s)
    if not first:
      wait_scatter(t)
    scatter(s)

  issue_loads(0, 0)
  issue_loads(1, 1)
  phase(0, 0, True)

  def chunk2(i, _):
    c = 2 * i + 1
    phase(c, 1, False)
    phase(c + 1, 0, False)
    return 0

  lax.fori_loop(0, (NCHUNK - 1) // 2, chunk2, 0)
  wait_scatter(0)
  wait_loads(NCHUNK, 1)
  wait_loads(NCHUNK + 1, 0)
  plsc.subcore_barrier()

  @pl.when(cid == 0)
  def _():
    pltpu.sync_copy(accum.at[pl.ds(sid * RPT, RPT)],
                    out0_hbm.at[pl.ds(sid * RPT, RPT)])

  @pl.when(cid == 1)
  def _():
    pltpu.sync_copy(accum.at[pl.ds(sid * RPT, RPT)],
                    out1_hbm.at[pl.ds(sid * RPT, RPT)])


_NPW = N_PAD // NW


def _dis_body(deg0_hbm, deg1_hbm, dis_hbm, s0, s1, dv):
  cid = lax.axis_index("c")
  sid = lax.axis_index("s")
  wid = cid * NS + sid
  base = wid * _NPW
  pltpu.sync_copy(deg0_hbm.at[pl.ds(base, _NPW)], s0)
  pltpu.sync_copy(deg1_hbm.at[pl.ds(base, _NPW)], s1)
  half = jnp.full((16,), 0.5, jnp.float32)
  three_half = jnp.full((16,), 1.5, jnp.float32)
  magic = jnp.full((16,), 0x5F3759DF, jnp.int32)
  fzero = jnp.zeros((16,), jnp.float32)

  def grp(g, _):
    sl = pl.ds(g * 16, 16)
    d = s0[sl] + s1[sl]
    y = lax.bitcast_convert_type(
        magic - (lax.bitcast_convert_type(d, jnp.int32) >> 1), jnp.float32)
    hx = d * half
    y = y * (three_half - hx * y * y)
    y = y * (three_half - hx * y * y)
    y = y * (three_half - hx * y * y)
    dv[sl] = jnp.where(d > fzero, y, fzero)
    return 0

  lax.fori_loop(0, _NPW // 16, grp, 0)
  pltpu.sync_copy(dv, dis_hbm.at[pl.ds(base, _NPW)])


def _spmm_body(row_hbm, col_hbm, dis_hbm, p_hbm, xg_hbm,
               out0_hbm, out1_hbm,
               dis_s, p_v, row_v, col_v, scol_v, dr_v, dc_v, gbuf, zbuf,
               accum, gsem0, gsem1, ssem0, ssem1):
  cid = lax.axis_index("c")
  sid = lax.axis_index("s")
  wid = cid * NS + sid
  zero = jnp.zeros((16,), jnp.float32)

  def zrow(i, _):
    for j in range(F // 16):
      zbuf[i, pl.ds(j * 16, 16)] = zero
    return 0

  lax.fori_loop(0, 128, zrow, 0)
  for t in range(RPT // 128):
    pltpu.sync_copy(zbuf, accum.at[pl.ds(sid * RPT + t * 128, 128)])

  @pl.when(sid == 0)
  def _():
    pltpu.sync_copy(dis_hbm, dis_s)

  pltpu.sync_copy(p_hbm, p_v)
  plsc.subcore_barrier()

  pvec = p_v[...]
  a0 = jnp.broadcast_to(lax.slice(pvec, (0,), (1,)), (16,))
  a1 = jnp.broadcast_to(lax.slice(pvec, (1,), (2,)), (16,))
  bb = jnp.broadcast_to(lax.slice(pvec, (2,), (3,)), (16,))
  mu = jnp.broadcast_to(lax.slice(pvec, (3,), (4,)), (16,))
  c2 = jnp.broadcast_to(lax.slice(pvec, (4,), (5,)), (16,))
  one = jnp.ones((16,), jnp.float32)
  two = jnp.full((16,), 2.0, jnp.float32)

  ebase = wid * EPT
  gsems = (gsem0, gsem1)
  ssems = (ssem0, ssem1)

  def load_idx(c, p):
    base = ebase + c * CH
    pltpu.sync_copy(row_hbm.at[pl.ds(base, CH)], row_v.at[p])
    pltpu.sync_copy(col_hbm.at[pl.ds(base, CH)], col_v.at[p])

  def issue_gather(p):
    pltpu.async_copy(xg_hbm.at[row_v.at[p]], gbuf.at[p], gsems[p])

  def wait_gather(p):
    pltpu.make_async_copy(xg_hbm.at[row_v.at[p]], gbuf.at[p],
                          gsems[p]).wait()

  def load_dis(p):
    pltpu.sync_copy(dis_s.at[row_v.at[p]], dr_v.at[p])
    pltpu.sync_copy(dis_s.at[col_v.at[p]], dc_v.at[p])

  def stash_col(p):
    for g in range(CH // 16):
      sl = pl.ds(g * 16, 16)
      scol_v[p, sl] = col_v[p, sl]

  def scatter(p):
    pltpu.async_copy(gbuf.at[p], accum.at[scol_v.at[p]], ssems[p], add=True)

  def wait_scatter(p):
    pltpu.make_async_copy(gbuf.at[p], accum.at[scol_v.at[p]],
                          ssems[p]).wait()

  def compute(p):
    for g in range(CH // 16):
      x = a0 * dr_v[p, pl.ds(g * 16, 16)] + \
          a1 * dc_v[p, pl.ds(g * 16, 16)] + bb
      t = one - two / (jnp.exp(x + x) + one)
      d = t - mu
      w = jnp.exp(c2 * d * d)
      for i in range(16):
        wb = jnp.broadcast_to(lax.slice(w, (i,), (i + 1,)), (16,))
        r = g * 16 + i
        for j in range(F // 16):
          sl = pl.ds(j * 16, 16)
          gbuf[p, r, sl] = gbuf[p, r, sl] * wb

  def phase(c, s, first):
    t = 1 - s
    load_idx(c + 1, t)
    load_dis(t)
    wait_gather(s)
    stash_col(s)
    if not first:
      wait_scatter(t)
    issue_gather(t)
    compute(s)
    scatter(s)

  load_idx(0, 0)
  load_dis(0)
  issue_gather(0)
  phase(0, 0, True)

  def chunk2(i, _):
    c = 2 * i + 1
    phase(c, 1, False)
    phase(c + 1, 0, False)
    return 0

  lax.fori_loop(0, (NCHUNK - 1) // 2, chunk2, 0)
  wait_scatter(0)
  wait_gather(1)
  plsc.subcore_barrier()

  @pl.when(cid == 0)
  def _():
    pltpu.sync_copy(accum.at[pl.ds(sid * RPT, RPT)],
                    out0_hbm.at[pl.ds(sid * RPT, RPT)])

  @pl.when(cid == 1)
  def _():
    pltpu.sync_copy(accum.at[pl.ds(sid * RPT, RPT)],
                    out1_hbm.at[pl.ds(sid * RPT, RPT)])


@functools.lru_cache(maxsize=1)
def _sc_kernels():
  mesh = plsc.VectorSubcoreMesh(**_MESH)
  deg_k = pl.kernel(
      _deg_body,
      out_type=[jax.ShapeDtypeStruct((N_PAD,), jnp.float32)] * 2,
      mesh=mesh,
      scratch_types=[
          pltpu.VMEM((RPT,), jnp.float32),
          pltpu.VMEM((2, CH), jnp.int32),
          pltpu.VMEM((2, CH), jnp.float32),
          pltpu.VMEM((2, CH), jnp.int32),
          pltpu.VMEM((2, CH), jnp.float32),
          pltpu.VMEM_SHARED((N_PAD,), jnp.float32),
          pltpu.SemaphoreType.DMA,
          pltpu.SemaphoreType.DMA,
          pltpu.SemaphoreType.DMA,
          pltpu.SemaphoreType.DMA,
          pltpu.SemaphoreType.DMA,
          pltpu.SemaphoreType.DMA,
      ],
  )
  dis_k = pl.kernel(
      _dis_body,
      out_type=jax.ShapeDtypeStruct((N_PAD,), jnp.float32),
      mesh=mesh,
      scratch_types=[
          pltpu.VMEM((_NPW,), jnp.float32),
          pltpu.VMEM((_NPW,), jnp.float32),
          pltpu.VMEM((_NPW,), jnp.float32),
      ],
  )
  spmm_k = pl.kernel(
      _spmm_body,
      out_type=[jax.ShapeDtypeStruct((N_PAD, F), jnp.float32)] * 2,
      mesh=mesh,
      scratch_types=[
          pltpu.VMEM_SHARED((N_PAD,), jnp.float32),
          pltpu.VMEM((16,), jnp.float32),
          pltpu.VMEM((2, CH), jnp.int32),
          pltpu.VMEM((2, CH), jnp.int32),
          pltpu.VMEM((2, CH), jnp.int32),
          pltpu.VMEM((2, CH), jnp.float32),
          pltpu.VMEM((2, CH), jnp.float32),
          pltpu.VMEM((2, CH, F), jnp.float32),
          pltpu.VMEM((128, F), jnp.float32),
          pltpu.VMEM_SHARED((N_PAD, F), jnp.float32),
          pltpu.SemaphoreType.DMA,
          pltpu.SemaphoreType.DMA,
          pltpu.SemaphoreType.DMA,
          pltpu.SemaphoreType.DMA,
      ],
  )
  return deg_k, dis_k, spmm_k


def _params_vec(Wp, bp, mu, sigma):
  a0 = Wp[0, 0]
  a1 = Wp[1, 0]
  b = bp[0]
  m = mu[0, 0]
  c2 = -0.5 / (EPS + sigma[0, 0] ** 2)
  return jnp.stack([a0, a1, b, m, c2,
                    0., 0., 0., 0., 0., 0., 0., 0., 0., 0., 0.]
                   ).astype(jnp.float32)


def kernel(h, edge_index, edge_weight, W_emb, b_emb, Wp0, bp0, Wg0, mu0,
           sigma0, root0, bias0, Wp1, bp1, Wg1, mu1, sigma1, root1, bias1,
           Wm, bm):
  row = jnp.pad(edge_index[0], (0, 2 * CH))
  col = jnp.pad(edge_index[1], (0, 2 * CH))
  ew = jnp.pad(edge_weight, (0, 2 * CH))
  _deg_kernel, _dis_kernel, _spmm_kernel = _sc_kernels()

  deg0, deg1 = _deg_kernel(row, ew)
  dis = _dis_kernel(deg0, deg1)

  p0 = _params_vec(Wp0, bp0, mu0, sigma0)
  p1 = _params_vec(Wp1, bp1, mu1, sigma1)

  m0 = pl.pallas_call(
      _m0_body,
      grid=(GRID,),
      in_specs=[_row_spec(), _full_spec((F, F)), _full_spec((1, F)),
                _full_spec((F, F)), _full_spec((F, F)), _full_spec((1, F))],
      out_specs=[_row_spec(), _row_spec(), _row_spec()],
      out_shape=[jax.ShapeDtypeStruct((N, F), jnp.float32)] * 3,
  )
  h1, xg0, hr0 = m0(h, W_emb, b_emb.reshape(1, F), Wg0, root0,
                    bias0.reshape(1, F))

  a0p0, a0p1 = _spmm_kernel(row, col, dis, p0, xg0)

  m1 = pl.pallas_call(
      _m1_body,
      grid=(GRID,),
      in_specs=[_row_spec(), _row_spec(), _row_spec(), _row_spec(),
                _full_spec((F, F)), _full_spec((F, F)), _full_spec((1, F))],
      out_specs=[_row_spec(), _row_spec(), _row_spec()],
      out_shape=[jax.ShapeDtypeStruct((N, F), jnp.float32)] * 3,
  )
  h2, xg1, hr1 = m1(h1, a0p0, a0p1, hr0, Wg1, root1, bias1.reshape(1, F))

  a1p0, a1p1 = _spmm_kernel(row, col, dis, p1, xg1)

  Wm_p = jnp.pad(Wm, ((0, 0), (0, F - Wm.shape[1])))
  bm_p = jnp.pad(bm, (0, F - bm.shape[0])).reshape(1, F)
  m2 = pl.pallas_call(
      _m2_body,
      grid=(GRID,),
      in_specs=[_row_spec(), _row_spec(), _row_spec(), _row_spec(),
                _full_spec((F, F)), _full_spec((1, F))],
      out_specs=_row_spec(),
      out_shape=jax.ShapeDtypeStruct((N, F), jnp.float32),
  )
  out = m2(h2, a1p0, a1p1, hr1, Wm_p, bm_p)
  return out[:, :Wm.shape[1]]


# speedup vs baseline: 20.7446x; 1.2730x over previous
import functools

import jax
import jax.numpy as jnp
from jax import lax
from jax.experimental import pallas as pl
from jax.experimental.pallas import tpu as pltpu
from jax.experimental.pallas import tpu_sc as plsc

N = 10000
E = 320000
F = 128
EPS = 1e-15

NC = 2
NS = 16
NW = NC * NS

N_PAD = 10240
RPT = N_PAD // NS
EPT = E // NW
CH = 80
NCHUNK = EPT // CH
DEG_LANES = 16

_MESH = dict(core_axis_name="c", subcore_axis_name="s", num_cores=NC,
             num_subcores=NS)

BLK = 2000
GRID = N // BLK


def _m0_body(h_ref, wemb_ref, bemb_ref, wg_ref, root_ref, bias_ref,
             h1_ref, xg_ref, hr_ref):
  h1 = jnp.dot(h_ref[...], wemb_ref[...],
               preferred_element_type=jnp.float32) + bemb_ref[...]
  h1_ref[...] = h1
  xg_ref[...] = jnp.dot(h1, wg_ref[...], preferred_element_type=jnp.float32)
  hr_ref[...] = jnp.dot(h1, root_ref[...],
                        preferred_element_type=jnp.float32) + bias_ref[...]


def _m1_body(h1_ref, ap0_ref, ap1_ref, hr_ref, wg_ref, root_ref, bias_ref,
             h2_ref, xg_ref, hr1_ref):
  aggr = ap0_ref[...] + ap1_ref[...]
  h2 = h1_ref[...] + jax.nn.relu(aggr + hr_ref[...])
  h2_ref[...] = h2
  xg_ref[...] = jnp.dot(h2, wg_ref[...], preferred_element_type=jnp.float32)
  hr1_ref[...] = jnp.dot(h2, root_ref[...],
                         preferred_element_type=jnp.float32) + bias_ref[...]


def _m2_body(h2_ref, ap0_ref, ap1_ref, hr_ref, wm_ref, bm_ref, o_ref):
  aggr = ap0_ref[...] + ap1_ref[...]
  h3 = h2_ref[...] + jax.nn.relu(aggr + hr_ref[...])
  o_ref[...] = jnp.dot(h3, wm_ref[...],
                       preferred_element_type=jnp.float32) + bm_ref[...]


def _row_spec():
  return pl.BlockSpec((BLK, F), lambda i: (i, 0))


def _full_spec(shape):
  nd = len(shape)
  return pl.BlockSpec(shape, lambda i: (0,) * nd)


def _part_spec():
  return pl.BlockSpec((NC, BLK, F), lambda i: (0, i, 0))


def _deg_body(row_hbm, ew_hbm, out0_hbm, out1_hbm, zbuf, row_v, ew_v,
              srow_v, sew_v, accum, rsem0, rsem1, esem0, esem1,
              ssem0, ssem1):
  cid = lax.axis_index("c")
  sid = lax.axis_index("s")
  wid = cid * NS + sid
  zero = jnp.zeros((16,), jnp.float32)

  def zrow(i, _):
    zbuf[pl.ds(i * 16, 16)] = zero
    return 0

  lax.fori_loop(0, RPT // 16, zrow, 0)
  pltpu.sync_copy(zbuf, accum.at[pl.ds(sid * RPT, RPT)])
  plsc.subcore_barrier()

  ebase = wid * EPT
  rsems = (rsem0, rsem1)
  esems = (esem0, esem1)
  ssems = (ssem0, ssem1)

  def issue_loads(c, p):
    base = ebase + c * CH
    pltpu.async_copy(row_hbm.at[pl.ds(base, CH)], row_v.at[p], rsems[p])
    pltpu.async_copy(ew_hbm.at[pl.ds(base, CH)], ew_v.at[p], esems[p])

  def wait_loads(c, p):
    base = ebase + c * CH
    pltpu.make_async_copy(row_hbm.at[pl.ds(base, CH)], row_v.at[p],
                          rsems[p]).wait()
    pltpu.make_async_copy(ew_hbm.at[pl.ds(base, CH)], ew_v.at[p],
                          esems[p]).wait()

  def stash(p):
    for g in range(CH // 16):
      sl = pl.ds(g * 16, 16)
      srow_v[p, sl] = row_v[p, sl]
      sew_v[p, sl] = ew_v[p, sl]

  def scatter(p):
    pltpu.async_copy(sew_v.at[p], accum.at[srow_v.at[p]], ssems[p],
                     add=True)

  def wait_scatter(p):
    pltpu.make_async_copy(sew_v.at[p], accum.at[srow_v.at[p]],
                          ssems[p]).wait()

  def phase(c, s, first):
    t = 1 - s
    wait_loads(c, s)
    stash(s)
    issue_loads(c + 2, s)
    if not first:
      wait_scatter(t)
    scatter(s)

  issue_loads(0, 0)
  issue_loads(1, 1)
  phase(0, 0, True)

  def chunk2(i, _):
    c = 2 * i + 1
    phase(c, 1, False)
    phase(c + 1, 0, False)
    return 0

  lax.fori_loop(0, (NCHUNK - 1) // 2, chunk2, 0)
  wait_scatter(0)
  wait_loads(NCHUNK, 1)
  wait_loads(NCHUNK + 1, 0)
  plsc.subcore_barrier()

  @pl.when(cid == 0)
  def _():
    pltpu.sync_copy(accum.at[pl.ds(sid * RPT, RPT)],
                    out0_hbm.at[pl.ds(sid * RPT, RPT)])

  @pl.when(cid == 1)
  def _():
    pltpu.sync_copy(accum.at[pl.ds(sid * RPT, RPT)],
                    out1_hbm.at[pl.ds(sid * RPT, RPT)])


_NPW = N_PAD // NW


def _dis_body(deg0_hbm, deg1_hbm, dis_hbm, s0, s1, dv):
  cid = lax.axis_index("c")
  sid = lax.axis_index("s")
  wid = cid * NS + sid
  base = wid * _NPW
  pltpu.sync_copy(deg0_hbm.at[pl.ds(base, _NPW)], s0)
  pltpu.sync_copy(deg1_hbm.at[pl.ds(base, _NPW)], s1)
  half = jnp.full((16,), 0.5, jnp.float32)
  three_half = jnp.full((16,), 1.5, jnp.float32)
  magic = jnp.full((16,), 0x5F3759DF, jnp.int32)
  fzero = jnp.zeros((16,), jnp.float32)

  def grp(g, _):
    sl = pl.ds(g * 16, 16)
    d = s0[sl] + s1[sl]
    y = lax.bitcast_convert_type(
        magic - (lax.bitcast_convert_type(d, jnp.int32) >> 1), jnp.float32)
    hx = d * half
    y = y * (three_half - hx * y * y)
    y = y * (three_half - hx * y * y)
    y = y * (three_half - hx * y * y)
    dv[sl] = jnp.where(d > fzero, y, fzero)
    return 0

  lax.fori_loop(0, _NPW // 16, grp, 0)
  pltpu.sync_copy(dv, dis_hbm.at[pl.ds(base, _NPW)])


def _spmm_body(row_hbm, col_hbm, dis_hbm, p_hbm, xg_hbm,
               out0_hbm, out1_hbm,
               dis_s, p_v, row_v, col_v, scol_v, dr_v, dc_v, gbuf, zbuf,
               accum, rsem0, rsem1, csem0, csem1, drsem0, drsem1,
               dcsem0, dcsem1, gsem0, gsem1, ssem0, ssem1):
  cid = lax.axis_index("c")
  sid = lax.axis_index("s")
  wid = cid * NS + sid
  zero = jnp.zeros((16,), jnp.float32)

  def zrow(i, _):
    for j in range(F // 16):
      zbuf[i, pl.ds(j * 16, 16)] = zero
    return 0

  lax.fori_loop(0, 128, zrow, 0)
  for t in range(RPT // 128):
    pltpu.sync_copy(zbuf, accum.at[pl.ds(sid * RPT + t * 128, 128)])

  @pl.when(sid == 0)
  def _():
    pltpu.sync_copy(dis_hbm, dis_s)

  pltpu.sync_copy(p_hbm, p_v)
  plsc.subcore_barrier()

  pvec = p_v[...]
  a0 = jnp.broadcast_to(lax.slice(pvec, (0,), (1,)), (16,))
  a1 = jnp.broadcast_to(lax.slice(pvec, (1,), (2,)), (16,))
  bb = jnp.broadcast_to(lax.slice(pvec, (2,), (3,)), (16,))
  mu = jnp.broadcast_to(lax.slice(pvec, (3,), (4,)), (16,))
  c2 = jnp.broadcast_to(lax.slice(pvec, (4,), (5,)), (16,))
  one = jnp.ones((16,), jnp.float32)
  two = jnp.full((16,), 2.0, jnp.float32)

  ebase = wid * EPT
  rsems = (rsem0, rsem1)
  csems = (csem0, csem1)
  drsems = (drsem0, drsem1)
  dcsems = (dcsem0, dcsem1)
  gsems = (gsem0, gsem1)
  ssems = (ssem0, ssem1)

  def issue_idx(c, p):
    base = ebase + c * CH
    pltpu.async_copy(row_hbm.at[pl.ds(base, CH)], row_v.at[p], rsems[p])
    pltpu.async_copy(col_hbm.at[pl.ds(base, CH)], col_v.at[p], csems[p])

  def wait_idx(c, p):
    base = ebase + c * CH
    pltpu.make_async_copy(row_hbm.at[pl.ds(base, CH)], row_v.at[p],
                          rsems[p]).wait()
    pltpu.make_async_copy(col_hbm.at[pl.ds(base, CH)], col_v.at[p],
                          csems[p]).wait()

  def issue_gathers(p):
    pltpu.async_copy(xg_hbm.at[row_v.at[p]], gbuf.at[p], gsems[p])
    pltpu.async_copy(dis_s.at[row_v.at[p]], dr_v.at[p], drsems[p])
    pltpu.async_copy(dis_s.at[col_v.at[p]], dc_v.at[p], dcsems[p])

  def wait_gathers(p):
    pltpu.make_async_copy(xg_hbm.at[row_v.at[p]], gbuf.at[p],
                          gsems[p]).wait()
    pltpu.make_async_copy(dis_s.at[row_v.at[p]], dr_v.at[p],
                          drsems[p]).wait()
    pltpu.make_async_copy(dis_s.at[col_v.at[p]], dc_v.at[p],
                          dcsems[p]).wait()

  def stash_col(p):
    for g in range(CH // 16):
      sl = pl.ds(g * 16, 16)
      scol_v[p, sl] = col_v[p, sl]

  def scatter(p):
    pltpu.async_copy(gbuf.at[p], accum.at[scol_v.at[p]], ssems[p], add=True)

  def wait_scatter(p):
    pltpu.make_async_copy(gbuf.at[p], accum.at[scol_v.at[p]],
                          ssems[p]).wait()

  def compute(p):
    for g in range(CH // 16):
      x = a0 * dr_v[p, pl.ds(g * 16, 16)] + \
          a1 * dc_v[p, pl.ds(g * 16, 16)] + bb
      t = one - two / (jnp.exp(x + x) + one)
      d = t - mu
      w = jnp.exp(c2 * d * d)
      for i in range(16):
        wb = jnp.broadcast_to(lax.slice(w, (i,), (i + 1,)), (16,))
        r = g * 16 + i
        for j in range(F // 16):
          sl = pl.ds(j * 16, 16)
          gbuf[p, r, sl] = gbuf[p, r, sl] * wb

  def phase(c, s, first):
    t = 1 - s
    wait_gathers(s)
    stash_col(s)
    issue_idx(c + 2, s)
    if not first:
      wait_scatter(t)
    wait_idx(c + 1, t)
    issue_gathers(t)
    compute(s)
    scatter(s)

  issue_idx(0, 0)
  issue_idx(1, 1)
  wait_idx(0, 0)
  issue_gathers(0)
  phase(0, 0, True)

  def chunk2(i, _):
    c = 2 * i + 1
    phase(c, 1, False)
    phase(c + 1, 0, False)
    return 0

  lax.fori_loop(0, (NCHUNK - 1) // 2, chunk2, 0)
  wait_scatter(0)
  wait_gathers(1)
  wait_idx(NCHUNK + 1, 0)
  plsc.subcore_barrier()

  @pl.when(cid == 0)
  def _():
    pltpu.sync_copy(accum.at[pl.ds(sid * RPT, RPT)],
                    out0_hbm.at[pl.ds(sid * RPT, RPT)])

  @pl.when(cid == 1)
  def _():
    pltpu.sync_copy(accum.at[pl.ds(sid * RPT, RPT)],
                    out1_hbm.at[pl.ds(sid * RPT, RPT)])


@functools.lru_cache(maxsize=1)
def _sc_kernels():
  mesh = plsc.VectorSubcoreMesh(**_MESH)
  deg_k = pl.kernel(
      _deg_body,
      out_type=[jax.ShapeDtypeStruct((N_PAD,), jnp.float32)] * 2,
      mesh=mesh,
      scratch_types=[
          pltpu.VMEM((RPT,), jnp.float32),
          pltpu.VMEM((2, CH), jnp.int32),
          pltpu.VMEM((2, CH), jnp.float32),
          pltpu.VMEM((2, CH), jnp.int32),
          pltpu.VMEM((2, CH), jnp.float32),
          pltpu.VMEM_SHARED((N_PAD,), jnp.float32),
          pltpu.SemaphoreType.DMA,
          pltpu.SemaphoreType.DMA,
          pltpu.SemaphoreType.DMA,
          pltpu.SemaphoreType.DMA,
          pltpu.SemaphoreType.DMA,
          pltpu.SemaphoreType.DMA,
      ],
  )
  dis_k = pl.kernel(
      _dis_body,
      out_type=jax.ShapeDtypeStruct((N_PAD,), jnp.float32),
      mesh=mesh,
      scratch_types=[
          pltpu.VMEM((_NPW,), jnp.float32),
          pltpu.VMEM((_NPW,), jnp.float32),
          pltpu.VMEM((_NPW,), jnp.float32),
      ],
  )
  spmm_k = pl.kernel(
      _spmm_body,
      out_type=[jax.ShapeDtypeStruct((N_PAD, F), jnp.float32)] * 2,
      mesh=mesh,
      scratch_types=[
          pltpu.VMEM_SHARED((N_PAD,), jnp.float32),
          pltpu.VMEM((16,), jnp.float32),
          pltpu.VMEM((2, CH), jnp.int32),
          pltpu.VMEM((2, CH), jnp.int32),
          pltpu.VMEM((2, CH), jnp.int32),
          pltpu.VMEM((2, CH), jnp.float32),
          pltpu.VMEM((2, CH), jnp.float32),
          pltpu.VMEM((2, CH, F), jnp.float32),
          pltpu.VMEM((128, F), jnp.float32),
          pltpu.VMEM_SHARED((N_PAD, F), jnp.float32),
          pltpu.SemaphoreType.DMA,
          pltpu.SemaphoreType.DMA,
          pltpu.SemaphoreType.DMA,
          pltpu.SemaphoreType.DMA,
          pltpu.SemaphoreType.DMA,
          pltpu.SemaphoreType.DMA,
          pltpu.SemaphoreType.DMA,
          pltpu.SemaphoreType.DMA,
          pltpu.SemaphoreType.DMA,
          pltpu.SemaphoreType.DMA,
          pltpu.SemaphoreType.DMA,
          pltpu.SemaphoreType.DMA,
      ],
  )
  return deg_k, dis_k, spmm_k


def _params_vec(Wp, bp, mu, sigma):
  a0 = Wp[0, 0]
  a1 = Wp[1, 0]
  b = bp[0]
  m = mu[0, 0]
  c2 = -0.5 / (EPS + sigma[0, 0] ** 2)
  return jnp.stack([a0, a1, b, m, c2,
                    0., 0., 0., 0., 0., 0., 0., 0., 0., 0., 0.]
                   ).astype(jnp.float32)


def kernel(h, edge_index, edge_weight, W_emb, b_emb, Wp0, bp0, Wg0, mu0,
           sigma0, root0, bias0, Wp1, bp1, Wg1, mu1, sigma1, root1, bias1,
           Wm, bm):
  row = jnp.pad(edge_index[0], (0, 2 * CH))
  col = jnp.pad(edge_index[1], (0, 2 * CH))
  ew = jnp.pad(edge_weight, (0, 2 * CH))
  _deg_kernel, _dis_kernel, _spmm_kernel = _sc_kernels()

  deg0, deg1 = _deg_kernel(row, ew)
  dis = _dis_kernel(deg0, deg1)

  p0 = _params_vec(Wp0, bp0, mu0, sigma0)
  p1 = _params_vec(Wp1, bp1, mu1, sigma1)

  m0 = pl.pallas_call(
      _m0_body,
      grid=(GRID,),
      in_specs=[_row_spec(), _full_spec((F, F)), _full_spec((1, F)),
                _full_spec((F, F)), _full_spec((F, F)), _full_spec((1, F))],
      out_specs=[_row_spec(), _row_spec(), _row_spec()],
      out_shape=[jax.ShapeDtypeStruct((N, F), jnp.float32)] * 3,
  )
  h1, xg0, hr0 = m0(h, W_emb, b_emb.reshape(1, F), Wg0, root0,
                    bias0.reshape(1, F))

  a0p0, a0p1 = _spmm_kernel(row, col, dis, p0, xg0)

  m1 = pl.pallas_call(
      _m1_body,
      grid=(GRID,),
      in_specs=[_row_spec(), _row_spec(), _row_spec(), _row_spec(),
                _full_spec((F, F)), _full_spec((F, F)), _full_spec((1, F))],
      out_specs=[_row_spec(), _row_spec(), _row_spec()],
      out_shape=[jax.ShapeDtypeStruct((N, F), jnp.float32)] * 3,
  )
  h2, xg1, hr1 = m1(h1, a0p0, a0p1, hr0, Wg1, root1, bias1.reshape(1, F))

  a1p0, a1p1 = _spmm_kernel(row, col, dis, p1, xg1)

  Wm_p = jnp.pad(Wm, ((0, 0), (0, F - Wm.shape[1])))
  bm_p = jnp.pad(bm, (0, F - bm.shape[0])).reshape(1, F)
  m2 = pl.pallas_call(
      _m2_body,
      grid=(GRID,),
      in_specs=[_row_spec(), _row_spec(), _row_spec(), _row_spec(),
                _full_spec((F, F)), _full_spec((1, F))],
      out_specs=_row_spec(),
      out_shape=jax.ShapeDtypeStruct((N, F), jnp.float32),
  )
  out = m2(h2, a1p0, a1p1, hr1, Wm_p, bm_p)
  return out[:, :Wm.shape[1]]
